# baseline (speedup 1.0000x reference)
import functools

import jax
import jax.numpy as jnp
from jax import lax
from jax.experimental import pallas as pl
from jax.experimental.pallas import tpu as pltpu
from jax.experimental.pallas import tpu_sc as plsc

_B, _L, _D = 4096, 50, 32
_NC, _NS = 2, 16
_NW = _NC * _NS
_BPW = _B // _NW
_NBUF = 4
_SCALE = 1.0 / _L

_mesh = plsc.VectorSubcoreMesh(
    core_axis_name="c", subcore_axis_name="s", num_cores=_NC, num_subcores=_NS
)


@functools.partial(
    pl.kernel,
    out_type=jax.ShapeDtypeStruct((_B, _D), jnp.float32),
    mesh=_mesh,
    scratch_types=[
        pltpu.VMEM((_BPW, _L), jnp.int32),
        pltpu.VMEM((_NBUF, _L, _D), jnp.float32),
        pltpu.VMEM((_BPW, _D), jnp.float32),
        pltpu.SemaphoreType.DMA((_NBUF,)),
    ],
    compiler_params=pltpu.CompilerParams(use_tc_tiling_on_sc=False),
)
def _lookup_mean(idx_hbm, table_hbm, out_hbm, idx_v, ring_v, out_v, sems):
    wid = lax.axis_index("s") * _NC + lax.axis_index("c")
    base = wid * _BPW
    pltpu.sync_copy(idx_hbm.at[pl.ds(base, _BPW)], idx_v)

    def _start(b, s):
        pltpu.async_copy(table_hbm.at[idx_v.at[b]], ring_v.at[s], sems.at[s])

    def _wait(b, s):
        pltpu.make_async_copy(
            table_hbm.at[idx_v.at[b]], ring_v.at[s], sems.at[s]
        ).wait()

    def _acc_row(b, s):
        _wait(b, s)
        a0 = ring_v[s, 0, 0:16]
        a1 = ring_v[s, 0, 16:32]
        for t in range(1, _L):
            a0 = a0 + ring_v[s, t, 0:16]
            a1 = a1 + ring_v[s, t, 16:32]
        return a0, a1

    for s in range(_NBUF):
        _start(s, s)

    @pl.loop(0, _BPW - _NBUF, step=_NBUF)
    def _main(b0):
        for s in range(_NBUF):
            b = b0 + s
            a0, a1 = _acc_row(b, s)
            _start(b + _NBUF, s)
            out_v[b, 0:16] = a0 * _SCALE
            out_v[b, 16:32] = a1 * _SCALE

    for s in range(_NBUF):
        b = (_BPW - _NBUF) + s
        a0, a1 = _acc_row(b, s)
        out_v[b, 0:16] = a0 * _SCALE
        out_v[b, 16:32] = a1 * _SCALE

    pltpu.sync_copy(out_v, out_hbm.at[pl.ds(base, _BPW)])


def kernel(idx, embedding):
    out = _lookup_mean(idx, embedding)
    return out[:, None, :]
